# baseline (speedup 1.0000x reference)
import functools

import jax
import jax.numpy as jnp
from jax import lax
from jax.experimental import pallas as pl
from jax.experimental.pallas import tpu as pltpu
from jax.experimental.pallas import tpu_sc as plsc

_B = 16384
_D = 64
_NC, _NS, _L = 2, 16, 16
_NW = _NC * _NS
_BPW = _B // _NW
_CHUNK = 128
_NCHUNK = _BPW // _CHUNK

_mesh = plsc.VectorSubcoreMesh(core_axis_name="c", subcore_axis_name="s")


@functools.partial(
    pl.kernel,
    out_type=jax.ShapeDtypeStruct((_NW, _L), jnp.float32),
    mesh=_mesh,
    scratch_types=[
        pltpu.VMEM((_NCHUNK, _CHUNK), jnp.int32),
        pltpu.VMEM((_BPW, _D), jnp.float32),
        pltpu.VMEM((_BPW, _D), jnp.float32),
        pltpu.VMEM((_L,), jnp.float32),
        pltpu.SemaphoreType.DMA,
        pltpu.SemaphoreType.DMA,
    ],
    compiler_params=pltpu.CompilerParams(use_tc_tiling_on_sc=False),
)
def _sc_partial(x_hbm, idx_hbm, tab_hbm, out_hbm,
                idx_v, g_v, x_v, acc_v, gsem, xsem):
    wid = lax.axis_index("c") * _NS + lax.axis_index("s")
    base = wid * _BPW

    x_copy = pltpu.async_copy(x_hbm.at[pl.ds(base, _BPW)], x_v, xsem)
    for j in range(_NCHUNK):
        pltpu.sync_copy(idx_hbm.at[pl.ds(base + j * _CHUNK, _CHUNK)],
                        idx_v.at[j])
    gathers = [
        pltpu.async_copy(tab_hbm.at[idx_v.at[j]],
                         g_v.at[pl.ds(j * _CHUNK, _CHUNK)], gsem)
        for j in range(_NCHUNK)
    ]
    x_copy.wait()
    for g in gathers:
        g.wait()

    def row_block(i, acc):
        r = i * 4
        for rr in range(4):
            for c in range(_D // _L):
                d = (x_v[r + rr, pl.ds(c * _L, _L)]
                     - g_v[r + rr, pl.ds(c * _L, _L)])
                acc = acc + d * d
        return acc

    acc = lax.fori_loop(0, _BPW // 4, row_block,
                        jnp.zeros((_L,), jnp.float32))
    acc_v[...] = acc
    pltpu.sync_copy(acc_v, out_hbm.at[wid])


def kernel(latent_X, cluster_id, clusters):
    idx = cluster_id.astype(jnp.int32)
    partials = _sc_partial(latent_X, idx, clusters)
    return 0.5 * jnp.sum(partials)


# speedup vs baseline: 1.1970x; 1.1970x over previous
import functools

import jax
import jax.numpy as jnp
from jax import lax
from jax.experimental import pallas as pl
from jax.experimental.pallas import tpu as pltpu
from jax.experimental.pallas import tpu_sc as plsc

_B = 16384
_D = 64
_NC, _NS, _L = 2, 16, 16
_NW = _NC * _NS
_BPW = _B // _NW
_CHUNK = 128
_NCHUNK = _BPW // _CHUNK

_mesh = plsc.VectorSubcoreMesh(core_axis_name="c", subcore_axis_name="s")


@functools.partial(
    pl.kernel,
    out_type=jax.ShapeDtypeStruct((_NW, _L), jnp.float32),
    mesh=_mesh,
    scratch_types=[
        pltpu.VMEM((_BPW,), jnp.int32),
        pltpu.VMEM((_BPW, _D), jnp.float32),
        pltpu.VMEM((_CHUNK, _D), jnp.float32),
        pltpu.VMEM((_L,), jnp.float32),
        pltpu.SemaphoreType.DMA,
        pltpu.SemaphoreType.DMA,
    ],
)
def _sc_partial(x_hbm, idx_hbm, tab_hbm, out_hbm,
                idx_s, g_v, x_v, acc_v, gsem, xsem):
    wid = lax.axis_index("c") * _NS + lax.axis_index("s")
    base = wid * _BPW

    pltpu.sync_copy(idx_hbm.at[pl.ds(base, _BPW)], idx_s)

    gathers = []
    for g in range(_BPW // _L):
        vec = idx_s[pl.ds(g * _L, _L)]
        for l in range(_L):
            gathers.append(
                pltpu.async_copy(tab_hbm.at[vec[l]], g_v.at[g * _L + l], gsem))
    acc = jnp.zeros((_L,), jnp.float32)
    for j in range(_NCHUNK):
        pltpu.sync_copy(x_hbm.at[pl.ds(base + j * _CHUNK, _CHUNK)], x_v)
        for g in gathers[j * _CHUNK:(j + 1) * _CHUNK]:
            g.wait()

        def row_block(i, acc, j=j):
            r = i * 4
            for rr in range(4):
                for c in range(_D // _L):
                    d = (x_v[r + rr, pl.ds(c * _L, _L)]
                         - g_v[j * _CHUNK + r + rr, pl.ds(c * _L, _L)])
                    acc = acc + d * d
            return acc

        acc = lax.fori_loop(0, _CHUNK // 4, row_block, acc)

    acc_v[...] = acc
    pltpu.sync_copy(acc_v, out_hbm.at[wid])


def kernel(latent_X, cluster_id, clusters):
    idx = cluster_id.astype(jnp.int32)
    partials = _sc_partial(latent_X, idx, clusters)
    return 0.5 * jnp.sum(partials)


# speedup vs baseline: 1.2007x; 1.0031x over previous
import functools

import jax
import jax.numpy as jnp
from jax import lax
from jax.experimental import pallas as pl
from jax.experimental.pallas import tpu as pltpu
from jax.experimental.pallas import tpu_sc as plsc

_B = 16384
_D = 64
_NC, _NS, _L = 2, 16, 16
_NW = _NC * _NS
_BPW = _B // _NW
_CHUNK = 128
_NCHUNK = _BPW // _CHUNK

_mesh = plsc.VectorSubcoreMesh(core_axis_name="c", subcore_axis_name="s")


@functools.partial(
    pl.kernel,
    out_type=jax.ShapeDtypeStruct((_NW, _L), jnp.float32),
    mesh=_mesh,
    scratch_types=[
        pltpu.VMEM((_BPW,), jnp.int32),
        pltpu.VMEM((_BPW, _D), jnp.float32),
        pltpu.VMEM((_CHUNK, _D), jnp.float32),
        pltpu.VMEM((_L,), jnp.float32),
        pltpu.SemaphoreType.DMA,
        pltpu.SemaphoreType.DMA,
    ],
    compiler_params=pltpu.CompilerParams(skip_device_barrier=True),
)
def _sc_partial(x_hbm, idx_hbm, tab_hbm, out_hbm,
                idx_s, g_v, x_v, acc_v, gsem, xsem):
    wid = lax.axis_index("c") * _NS + lax.axis_index("s")
    base = wid * _BPW

    pltpu.sync_copy(idx_hbm.at[pl.ds(base, _BPW)], idx_s)

    gathers = []
    for g in range(_BPW // _L):
        vec = idx_s[pl.ds(g * _L, _L)]
        for l in range(_L):
            gathers.append(
                pltpu.async_copy(tab_hbm.at[vec[l]], g_v.at[g * _L + l], gsem))
    acc = jnp.zeros((_L,), jnp.float32)
    for j in range(_NCHUNK):
        pltpu.sync_copy(x_hbm.at[pl.ds(base + j * _CHUNK, _CHUNK)], x_v)
        for g in gathers[j * _CHUNK:(j + 1) * _CHUNK]:
            g.wait()

        def row_block(i, acc, j=j):
            r = i * 4
            for rr in range(4):
                for c in range(_D // _L):
                    d = (x_v[r + rr, pl.ds(c * _L, _L)]
                         - g_v[j * _CHUNK + r + rr, pl.ds(c * _L, _L)])
                    acc = acc + d * d
            return acc

        acc = lax.fori_loop(0, _CHUNK // 4, row_block, acc)

    acc_v[...] = acc
    pltpu.sync_copy(acc_v, out_hbm.at[wid])


def kernel(latent_X, cluster_id, clusters):
    idx = cluster_id.astype(jnp.int32)
    partials = _sc_partial(latent_X, idx, clusters)
    return 0.5 * jnp.sum(partials)


# speedup vs baseline: 1.8069x; 1.5049x over previous
import functools

import jax
import jax.numpy as jnp
from jax import lax
from jax.experimental import pallas as pl
from jax.experimental.pallas import tpu as pltpu
from jax.experimental.pallas import tpu_sc as plsc

_B = 16384
_D = 64
_NC, _NS, _L = 2, 16, 16
_NW = _NC * _NS
_FPW = _D // _NW
_V = 100000
_XCHUNK = 4096
_NXCHUNK = _B // _XCHUNK

_mesh = plsc.VectorSubcoreMesh(core_axis_name="c", subcore_axis_name="s")


@functools.partial(
    pl.kernel,
    out_type=jax.ShapeDtypeStruct((_NW, _L), jnp.float32),
    mesh=_mesh,
    scratch_types=[
        pltpu.VMEM((_B,), jnp.int32),
        pltpu.VMEM((1, _V), jnp.float32),
        pltpu.VMEM((1, _XCHUNK), jnp.float32),
        pltpu.VMEM((_L,), jnp.float32),
        pltpu.SemaphoreType.DMA,
        pltpu.SemaphoreType.DMA,
    ],
    compiler_params=pltpu.CompilerParams(skip_device_barrier=True,
                                         needs_layout_passes=False),
)
def _sc_partial(xt_hbm, idx_hbm, tabt_hbm, out_hbm,
                idx_v, crow_v, x_v, acc_v, csem, xsem):
    wid = lax.axis_index("c") * _NS + lax.axis_index("s")
    f0 = wid * _FPW

    row_copy = pltpu.async_copy(tabt_hbm.at[f0], crow_v.at[0], csem)
    pltpu.sync_copy(idx_hbm, idx_v)
    zeros16 = jnp.zeros((_L,), jnp.int32)

    acc = jnp.zeros((_L,), jnp.float32)
    for fi in range(_FPW):
        row_copy.wait()
        for cx in range(_NXCHUNK):
            pltpu.sync_copy(xt_hbm.at[f0 + fi, pl.ds(cx * _XCHUNK, _XCHUNK)],
                            x_v.at[0])

            def group(g, acc, cx=cx):
                idxv = idx_v[pl.ds(cx * _XCHUNK + g * _L, _L)]
                cv = plsc.load_gather(crow_v, [zeros16, idxv])
                xv = x_v[0, pl.ds(g * _L, _L)]
                d = xv - cv
                return acc + d * d

            acc = lax.fori_loop(0, _XCHUNK // _L, group, acc)
        if fi + 1 < _FPW:
            row_copy = pltpu.async_copy(tabt_hbm.at[f0 + fi + 1],
                                        crow_v.at[0], csem)

    acc_v[...] = acc
    pltpu.sync_copy(acc_v, out_hbm.at[wid])


def kernel(latent_X, cluster_id, clusters):
    idx = cluster_id.astype(jnp.int32)
    partials = _sc_partial(latent_X.T, idx, clusters.T)
    return 0.5 * jnp.sum(partials)


# speedup vs baseline: 2.3318x; 1.2905x over previous
import functools

import jax
import jax.numpy as jnp
from jax import lax
from jax.experimental import pallas as pl
from jax.experimental.pallas import tpu as pltpu
from jax.experimental.pallas import tpu_sc as plsc

_B = 16384
_D = 64
_NC, _NS, _L = 2, 16, 16
_NW = _NC * _NS
_FPW = _D // _NW
_V = 100000
_XCHUNK = 4096
_NXCHUNK = _B // _XCHUNK

_mesh = plsc.VectorSubcoreMesh(core_axis_name="c", subcore_axis_name="s")


@functools.partial(
    pl.kernel,
    out_type=jax.ShapeDtypeStruct((_NW, _L), jnp.float32),
    mesh=_mesh,
    scratch_types=[
        pltpu.VMEM((_B,), jnp.int32),
        pltpu.VMEM((1, _V), jnp.float32),
        pltpu.VMEM((2, _XCHUNK), jnp.float32),
        pltpu.VMEM((_L,), jnp.float32),
        pltpu.SemaphoreType.DMA,
        pltpu.SemaphoreType.DMA,
    ],
    compiler_params=pltpu.CompilerParams(skip_device_barrier=True,
                                         needs_layout_passes=False),
)
def _sc_partial(xt_hbm, idx_hbm, tabt_hbm, out_hbm,
                idx_v, crow_v, x_v, acc_v, csem, xsem):
    wid = lax.axis_index("c") * _NS + lax.axis_index("s")
    f0 = wid * _FPW

    row_copy = pltpu.async_copy(tabt_hbm.at[f0], crow_v.at[0], csem)
    pltpu.sync_copy(idx_hbm, idx_v)
    crow_flat = crow_v.at[0]
    _UNROLL = 8

    acc = jnp.zeros((_L,), jnp.float32)
    for fi in range(_FPW):
        x_copies = [pltpu.async_copy(
            xt_hbm.at[f0 + fi, pl.ds(0, _XCHUNK)], x_v.at[0], xsem)]
        row_copy.wait()
        for cx in range(_NXCHUNK):
            if cx + 1 < _NXCHUNK:
                x_copies.append(pltpu.async_copy(
                    xt_hbm.at[f0 + fi, pl.ds((cx + 1) * _XCHUNK, _XCHUNK)],
                    x_v.at[(cx + 1) % 2], xsem))
            x_copies[cx].wait()
            xbuf = cx % 2

            def group(g, acc, cx=cx, xbuf=xbuf):
                o = g * _L
                idxv = idx_v[pl.ds(cx * _XCHUNK + o, _L)]
                cv = plsc.load_gather(crow_flat, [idxv])
                xv = x_v[xbuf, pl.ds(o, _L)]
                d = xv - cv
                return acc + d * d

            acc = plsc.parallel_loop(0, _XCHUNK // _L, step=1,
                                     unroll=_UNROLL, carry=acc)(group)
        if fi + 1 < _FPW:
            row_copy = pltpu.async_copy(tabt_hbm.at[f0 + fi + 1],
                                        crow_v.at[0], csem)

    acc_v[...] = acc
    pltpu.sync_copy(acc_v, out_hbm.at[wid])


def kernel(latent_X, cluster_id, clusters):
    idx = cluster_id.astype(jnp.int32)
    partials = _sc_partial(latent_X.T, idx, clusters.T)
    return 0.5 * jnp.sum(partials)
